# baseline (speedup 1.0000x reference)
import functools

import jax
import jax.numpy as jnp
from jax import lax
from jax.experimental import pallas as pl
from jax.experimental.pallas import tpu as pltpu
from jax.experimental.pallas import tpu_sc as plsc

N_FIELDS = 26
F_DIM = 100000
B_TOTAL = 16384
ROWS = N_FIELDS * F_DIM
HALF = ROWS // 2

_info = plsc.get_sparse_core_info()
NC, NS, L = _info.num_cores, _info.num_subcores, _info.num_lanes
NW = NC * NS
B_W = B_TOTAL // NW
IDX_W = B_W * N_FIELDS
N_DMA_ROWS = IDX_W // 128


def _wid():
    return lax.axis_index("s") * NC + lax.axis_index("c")


def _idx_body(xt_hbm, idx_hbm, x_v, idx_v):
    wid = _wid()

    pltpu.sync_copy(xt_hbm.at[:, pl.ds(wid * B_W, B_W)], x_v)

    lanes = lax.iota(jnp.int32, L)
    zeros = jnp.zeros((L,), jnp.int32)

    def off_f(f, _):
        f_vec = zeros + f
        for t in range(NW):
            vals = plsc.load_gather(x_v, [f_vec, t * L + lanes]) + f * F_DIM
            idx_v.at[f * 4 + (t // 8)][pl.ds((t % 8) * L, L)] = vals
        return 0

    lax.fori_loop(0, N_FIELDS, off_f, 0)

    pltpu.sync_copy(idx_v, idx_hbm.at[pl.ds(wid * N_DMA_ROWS, N_DMA_ROWS)])


def _gather_body(lo, hi, table_hbm, idx_hbm, out_hbm, idx_v, mask_v, rows_v,
                 out_v, sem):
    wid = _wid()

    pltpu.sync_copy(idx_hbm.at[pl.ds(wid * N_DMA_ROWS, N_DMA_ROWS)], idx_v)

    ones_f = jnp.ones((L,), jnp.float32)
    zeros_f = jnp.zeros((L,), jnp.float32)
    zeros_i = jnp.zeros((L,), jnp.int32)

    def prep(j, _):
        for c in range(8):
            sl = pl.ds(c * L, L)
            iv = idx_v.at[j][sl]
            m = (iv >= lo) & (iv < hi)
            idx_v.at[j][sl] = jnp.where(m, iv - lo, zeros_i)
            mask_v.at[j][sl] = jnp.where(m, ones_f, zeros_f)
        return 0

    lax.fori_loop(0, N_DMA_ROWS, prep, 0)

    def fire(j, _):
        pltpu.make_async_copy(
            table_hbm.at[idx_v.at[j]], rows_v.at[j], sem
        ).start()
        return 0

    lax.fori_loop(0, N_DMA_ROWS, fire, 0)

    def drain(j, _):
        pltpu.make_async_copy(
            table_hbm.at[idx_v.at[j]], rows_v.at[j], sem
        ).wait()
        return 0

    lax.fori_loop(0, N_DMA_ROWS, drain, 0)

    def reduce_t(t, _):
        g = t >> 3
        sl = pl.ds((t & 7) * L, L)
        acc = jnp.zeros((L,), jnp.float32)
        for f in range(N_FIELDS):
            r = f * 4 + g
            acc = acc + rows_v.at[r][sl] * mask_v.at[r][sl]
        out_v[pl.ds(t * L, L)] = acc
        return 0

    lax.fori_loop(0, NW, reduce_t, 0)

    pltpu.sync_copy(out_v, out_hbm.at[pl.ds(wid * B_W, B_W)])


@jax.jit
def _features_linear(x, table):
    mesh = plsc.VectorSubcoreMesh(core_axis_name="c", subcore_axis_name="s")
    params = pltpu.CompilerParams(needs_layout_passes=False)

    idx_all = pl.kernel(
        _idx_body,
        mesh=mesh,
        compiler_params=params,
        out_type=jax.ShapeDtypeStruct((NW * N_DMA_ROWS, 128), jnp.int32),
        scratch_types=[
            pltpu.VMEM((N_FIELDS, B_W), jnp.int32),
            pltpu.VMEM((N_DMA_ROWS, 128), jnp.int32),
        ],
    )(jnp.transpose(x, (1, 0)))

    def gather_half(flat_half, lo, hi):
        return pl.kernel(
            functools.partial(_gather_body, lo, hi),
            mesh=mesh,
            compiler_params=params,
            out_type=jax.ShapeDtypeStruct((B_TOTAL,), jnp.float32),
            scratch_types=[
                pltpu.VMEM((N_DMA_ROWS, 128), jnp.int32),
                pltpu.VMEM((N_DMA_ROWS, 128), jnp.float32),
                pltpu.VMEM((N_DMA_ROWS, 128), jnp.float32),
                pltpu.VMEM((B_W,), jnp.float32),
                pltpu.SemaphoreType.DMA,
            ],
        )(flat_half, idx_all)

    out0 = gather_half(table[:HALF, 0], 0, HALF)
    out1 = gather_half(table[HALF:, 0], HALF, ROWS)
    return out0 + out1


def kernel(x, table, bias):
    out = _features_linear(x, table)
    return out.reshape(B_TOTAL, 1) + bias


# speedup vs baseline: 1.0020x; 1.0020x over previous
import functools

import jax
import jax.numpy as jnp
from jax import lax
from jax.experimental import pallas as pl
from jax.experimental.pallas import tpu as pltpu
from jax.experimental.pallas import tpu_sc as plsc

N_FIELDS = 26
F_DIM = 100000
B_TOTAL = 16384
ROWS = N_FIELDS * F_DIM
HALF = (ROWS // 2 // 1024) * 1024

_info = plsc.get_sparse_core_info()
NC, NS, L = _info.num_cores, _info.num_subcores, _info.num_lanes
NW = NC * NS
B_W = B_TOTAL // NW
IDX_W = B_W * N_FIELDS
N_DMA_ROWS = IDX_W // 128


def _wid():
    return lax.axis_index("s") * NC + lax.axis_index("c")


def _idx_body(xt_hbm, idx_hbm, x_v, idx_v):
    wid = _wid()

    pltpu.sync_copy(xt_hbm.at[:, pl.ds(wid * B_W, B_W)], x_v)

    lanes = lax.iota(jnp.int32, L)
    zeros = jnp.zeros((L,), jnp.int32)

    def off_f(f, _):
        f_vec = zeros + f
        for t in range(NW):
            vals = plsc.load_gather(x_v, [f_vec, t * L + lanes]) + f * F_DIM
            idx_v.at[f * 4 + (t // 8)][pl.ds((t % 8) * L, L)] = vals
        return 0

    lax.fori_loop(0, N_FIELDS, off_f, 0)

    pltpu.sync_copy(idx_v, idx_hbm.at[pl.ds(wid * N_DMA_ROWS, N_DMA_ROWS)])


def _gather_body(lo, hi, table_hbm, idx_hbm, out_hbm, idx_v, mask_v, rows_v,
                 out_v, sem):
    wid = _wid()

    pltpu.sync_copy(idx_hbm.at[pl.ds(wid * N_DMA_ROWS, N_DMA_ROWS)], idx_v)

    ones_f = jnp.ones((L,), jnp.float32)
    zeros_f = jnp.zeros((L,), jnp.float32)
    zeros_i = jnp.zeros((L,), jnp.int32)

    def prep(j, _):
        for c in range(8):
            sl = pl.ds(c * L, L)
            iv = idx_v.at[j][sl]
            m = (iv >= lo) & (iv < hi)
            idx_v.at[j][sl] = jnp.where(m, iv - lo, zeros_i)
            mask_v.at[j][sl] = jnp.where(m, ones_f, zeros_f)
        return 0

    lax.fori_loop(0, N_DMA_ROWS, prep, 0)

    def fire(j, _):
        pltpu.make_async_copy(
            table_hbm.at[idx_v.at[j]], rows_v.at[j], sem
        ).start()
        return 0

    lax.fori_loop(0, N_DMA_ROWS, fire, 0)

    def drain(j, _):
        pltpu.make_async_copy(
            table_hbm.at[idx_v.at[j]], rows_v.at[j], sem
        ).wait()
        return 0

    lax.fori_loop(0, N_DMA_ROWS, drain, 0)

    def reduce_t(t, _):
        g = t >> 3
        sl = pl.ds((t & 7) * L, L)
        acc = jnp.zeros((L,), jnp.float32)
        for f in range(N_FIELDS):
            r = f * 4 + g
            acc = acc + rows_v.at[r][sl] * mask_v.at[r][sl]
        out_v[pl.ds(t * L, L)] = acc
        return 0

    lax.fori_loop(0, NW, reduce_t, 0)

    pltpu.sync_copy(out_v, out_hbm.at[pl.ds(wid * B_W, B_W)])


@jax.jit
def _features_linear(x, table):
    mesh = plsc.VectorSubcoreMesh(core_axis_name="c", subcore_axis_name="s")
    params = pltpu.CompilerParams(needs_layout_passes=False)

    idx_all = pl.kernel(
        _idx_body,
        mesh=mesh,
        compiler_params=params,
        out_type=jax.ShapeDtypeStruct((NW * N_DMA_ROWS, 128), jnp.int32),
        scratch_types=[
            pltpu.VMEM((N_FIELDS, B_W), jnp.int32),
            pltpu.VMEM((N_DMA_ROWS, 128), jnp.int32),
        ],
    )(jnp.transpose(x, (1, 0)))

    def gather_half(flat_half, lo, hi):
        return pl.kernel(
            functools.partial(_gather_body, lo, hi),
            mesh=mesh,
            compiler_params=params,
            out_type=jax.ShapeDtypeStruct((B_TOTAL,), jnp.float32),
            scratch_types=[
                pltpu.VMEM((N_DMA_ROWS, 128), jnp.int32),
                pltpu.VMEM((N_DMA_ROWS, 128), jnp.float32),
                pltpu.VMEM((N_DMA_ROWS, 128), jnp.float32),
                pltpu.VMEM((B_W,), jnp.float32),
                pltpu.SemaphoreType.DMA,
            ],
        )(flat_half, idx_all)

    out0 = gather_half(table[:HALF, 0], 0, HALF)
    out1 = gather_half(table[HALF:, 0], HALF, ROWS)
    return out0 + out1


def kernel(x, table, bias):
    out = _features_linear(x, table)
    return out.reshape(B_TOTAL, 1) + bias


# speedup vs baseline: 18.1562x; 18.1197x over previous
import functools

import jax
import jax.numpy as jnp
from jax import lax
from jax.experimental import pallas as pl
from jax.experimental.pallas import tpu as pltpu
from jax.experimental.pallas import tpu_sc as plsc

N_FIELDS = 26
F_DIM = 100000
B_TOTAL = 16384
ROWS = N_FIELDS * F_DIM
HALF = (ROWS // 2 // 1024) * 1024

_info = plsc.get_sparse_core_info()
NC, NS, L = _info.num_cores, _info.num_subcores, _info.num_lanes
NW = NC * NS
B_W = B_TOTAL // NW
IDX_W = B_W * N_FIELDS
N_DMA_ROWS = IDX_W // 128


def _wid():
    return lax.axis_index("s") * NC + lax.axis_index("c")


def _idx_body(xt_hbm, idx_hbm, x_v, idx_v):
    wid = _wid()

    pltpu.sync_copy(xt_hbm.at[:, pl.ds(wid * B_W, B_W)], x_v)

    lanes = lax.iota(jnp.int32, L)
    zeros = jnp.zeros((L,), jnp.int32)

    def off_f(f, _):
        f_vec = zeros + f
        for t in range(NW):
            vals = plsc.load_gather(x_v, [f_vec, t * L + lanes]) + f * F_DIM
            idx_v.at[f * 4 + (t // 8)][pl.ds((t % 8) * L, L)] = vals
        return 0

    lax.fori_loop(0, N_FIELDS, off_f, 0)

    pltpu.sync_copy(idx_v, idx_hbm.at[pl.ds(wid * N_DMA_ROWS, N_DMA_ROWS)])


def _gather_body(lo, hi, table_hbm, idx_hbm, out_hbm, idx_v, mask_v, rows_v,
                 out_v, sem):
    wid = _wid()

    pltpu.sync_copy(idx_hbm.at[pl.ds(wid * N_DMA_ROWS, N_DMA_ROWS)], idx_v)

    ones_f = jnp.ones((L,), jnp.float32)
    zeros_f = jnp.zeros((L,), jnp.float32)
    zeros_i = jnp.zeros((L,), jnp.int32)

    def prep(j, _):
        for c in range(8):
            sl = pl.ds(c * L, L)
            iv = idx_v.at[j][sl]
            if lo == 0:
                m = iv < hi
                alt = iv - hi
            else:
                m = iv >= lo
                alt = jnp.minimum(iv, hi - lo - 1)
            idx_v.at[j][sl] = jnp.where(m, iv - lo, alt)
            mask_v.at[j][sl] = jnp.where(m, ones_f, zeros_f)
        return 0

    lax.fori_loop(0, N_DMA_ROWS, prep, 0)

    def fire(j, _):
        pltpu.make_async_copy(
            table_hbm.at[idx_v.at[j]], rows_v.at[j], sem
        ).start()
        return 0

    lax.fori_loop(0, N_DMA_ROWS, fire, 0)

    def drain(j, _):
        pltpu.make_async_copy(
            table_hbm.at[idx_v.at[j]], rows_v.at[j], sem
        ).wait()
        return 0

    lax.fori_loop(0, N_DMA_ROWS, drain, 0)

    def reduce_t(t, _):
        g = t >> 3
        sl = pl.ds((t & 7) * L, L)
        acc = jnp.zeros((L,), jnp.float32)
        for f in range(N_FIELDS):
            r = f * 4 + g
            acc = acc + rows_v.at[r][sl] * mask_v.at[r][sl]
        out_v[pl.ds(t * L, L)] = acc
        return 0

    lax.fori_loop(0, NW, reduce_t, 0)

    pltpu.sync_copy(out_v, out_hbm.at[pl.ds(wid * B_W, B_W)])


@jax.jit
def _features_linear(x, table):
    mesh = plsc.VectorSubcoreMesh(core_axis_name="c", subcore_axis_name="s")
    params = pltpu.CompilerParams(needs_layout_passes=False)

    idx_all = pl.kernel(
        _idx_body,
        mesh=mesh,
        compiler_params=params,
        out_type=jax.ShapeDtypeStruct((NW * N_DMA_ROWS, 128), jnp.int32),
        scratch_types=[
            pltpu.VMEM((N_FIELDS, B_W), jnp.int32),
            pltpu.VMEM((N_DMA_ROWS, 128), jnp.int32),
        ],
    )(jnp.transpose(x, (1, 0)))

    def gather_half(flat_half, lo, hi):
        return pl.kernel(
            functools.partial(_gather_body, lo, hi),
            mesh=mesh,
            compiler_params=params,
            out_type=jax.ShapeDtypeStruct((B_TOTAL,), jnp.float32),
            scratch_types=[
                pltpu.VMEM((N_DMA_ROWS, 128), jnp.int32),
                pltpu.VMEM((N_DMA_ROWS, 128), jnp.float32),
                pltpu.VMEM((N_DMA_ROWS, 128), jnp.float32),
                pltpu.VMEM((B_W,), jnp.float32),
                pltpu.SemaphoreType.DMA,
            ],
        )(flat_half, idx_all)

    out0 = gather_half(table[:HALF, 0], 0, HALF)
    out1 = gather_half(table[HALF:, 0], HALF, ROWS)
    return out0 + out1


def kernel(x, table, bias):
    out = _features_linear(x, table)
    return out.reshape(B_TOTAL, 1) + bias


# speedup vs baseline: 27.3741x; 1.5077x over previous
import functools

import jax
import jax.numpy as jnp
from jax import lax
from jax.experimental import pallas as pl
from jax.experimental.pallas import tpu as pltpu
from jax.experimental.pallas import tpu_sc as plsc

N_FIELDS = 26
F_DIM = 100000
B_TOTAL = 16384
ROWS = N_FIELDS * F_DIM
P0_LEN = 1299456
P1_START = 1298496
P1_LEN = ROWS - P1_START
SPLIT = P0_LEN

_info = plsc.get_sparse_core_info()
NC, NS, L = _info.num_cores, _info.num_subcores, _info.num_lanes
NW = NC * NS
B_W = B_TOTAL // NW
IDX_W = B_W * N_FIELDS
N_DMA_ROWS = IDX_W // 128


def _wid():
    return lax.axis_index("s") * NC + lax.axis_index("c")


def _idx_body(xt_hbm, idx_hbm, x_v, idx_v):
    wid = _wid()

    pltpu.sync_copy(xt_hbm.at[:, pl.ds(wid * B_W, B_W)], x_v)

    lanes = lax.iota(jnp.int32, L)
    zeros = jnp.zeros((L,), jnp.int32)

    def off_f(f, _):
        f_vec = zeros + f
        for t in range(NW):
            vals = plsc.load_gather(x_v, [f_vec, t * L + lanes]) + f * F_DIM
            idx_v.at[f * 4 + (t // 8)][pl.ds((t % 8) * L, L)] = vals
        return 0

    lax.fori_loop(0, N_FIELDS, off_f, 0)

    pltpu.sync_copy(idx_v, idx_hbm.at[pl.ds(wid * N_DMA_ROWS, N_DMA_ROWS)])


def _gather_body(base, length, first, table_hbm, idx_hbm, out_hbm, idx_v,
                 mask_v, rows_v, out_v, sem):
    wid = _wid()

    pltpu.sync_copy(idx_hbm.at[pl.ds(wid * N_DMA_ROWS, N_DMA_ROWS)], idx_v)

    ones_f = jnp.ones((L,), jnp.float32)
    zeros_f = jnp.zeros((L,), jnp.float32)
    zeros_i = jnp.zeros((L,), jnp.int32)

    def prep(j, _):
        for c in range(8):
            sl = pl.ds(c * L, L)
            iv = idx_v.at[j][sl]
            if first:
                m = iv < SPLIT
                alt = jnp.minimum(iv - SPLIT, length - 1)
            else:
                m = iv >= SPLIT
                alt = jnp.minimum(iv, length - 1)
            idx_v.at[j][sl] = jnp.where(m, iv - base, alt)
            mask_v.at[j][sl] = jnp.where(m, ones_f, zeros_f)
        return 0

    lax.fori_loop(0, N_DMA_ROWS, prep, 0)

    def fire(j, _):
        pltpu.make_async_copy(
            table_hbm.at[idx_v.at[j]], rows_v.at[j], sem
        ).start()
        return 0

    lax.fori_loop(0, N_DMA_ROWS, fire, 0)

    def drain(j, _):
        pltpu.make_async_copy(
            table_hbm.at[idx_v.at[j]], rows_v.at[j], sem
        ).wait()
        return 0

    lax.fori_loop(0, N_DMA_ROWS, drain, 0)

    def reduce_t(t, _):
        g = t >> 3
        sl = pl.ds((t & 7) * L, L)
        acc = jnp.zeros((L,), jnp.float32)
        for f in range(N_FIELDS):
            r = f * 4 + g
            acc = acc + rows_v.at[r][sl] * mask_v.at[r][sl]
        out_v[pl.ds(t * L, L)] = acc
        return 0

    lax.fori_loop(0, NW, reduce_t, 0)

    pltpu.sync_copy(out_v, out_hbm.at[pl.ds(wid * B_W, B_W)])


@jax.jit
def _features_linear(x, table):
    mesh = plsc.VectorSubcoreMesh(core_axis_name="c", subcore_axis_name="s")
    params = pltpu.CompilerParams(needs_layout_passes=False)

    idx_all = pl.kernel(
        _idx_body,
        mesh=mesh,
        compiler_params=params,
        out_type=jax.ShapeDtypeStruct((NW * N_DMA_ROWS, 128), jnp.int32),
        scratch_types=[
            pltpu.VMEM((N_FIELDS, B_W), jnp.int32),
            pltpu.VMEM((N_DMA_ROWS, 128), jnp.int32),
        ],
    )(jnp.transpose(x, (1, 0)))

    def gather_half(flat_half, base, length, first):
        return pl.kernel(
            functools.partial(_gather_body, base, length, first),
            mesh=mesh,
            compiler_params=params,
            out_type=jax.ShapeDtypeStruct((B_TOTAL,), jnp.float32),
            scratch_types=[
                pltpu.VMEM((N_DMA_ROWS, 128), jnp.int32),
                pltpu.VMEM((N_DMA_ROWS, 128), jnp.float32),
                pltpu.VMEM((N_DMA_ROWS, 128), jnp.float32),
                pltpu.VMEM((B_W,), jnp.float32),
                pltpu.SemaphoreType.DMA,
            ],
        )(flat_half, idx_all)

    out0 = gather_half(table[:P0_LEN, 0], 0, P0_LEN, True)
    out1 = gather_half(table[P1_START:, 0], P1_START, P1_LEN, False)
    return out0 + out1


def kernel(x, table, bias):
    out = _features_linear(x, table)
    return out.reshape(B_TOTAL, 1) + bias


# speedup vs baseline: 28.9120x; 1.0562x over previous
import jax
import jax.numpy as jnp
from jax import lax
from jax.experimental import pallas as pl
from jax.experimental.pallas import tpu as pltpu
from jax.experimental.pallas import tpu_sc as plsc

N_FIELDS = 26
F_DIM = 100000
B_TOTAL = 16384
ROWS = N_FIELDS * F_DIM

P0_LEN = 1299456
P1_START = 1298496
P1_LEN = ROWS - P1_START
SPLIT = P0_LEN

_info = plsc.get_sparse_core_info()
NC, NS, L = _info.num_cores, _info.num_subcores, _info.num_lanes
NW = NC * NS
B_W = B_TOTAL // NW
IDX_W = B_W * N_FIELDS
N_DMA_ROWS = IDX_W // 128


def _wid():
    return lax.axis_index("s") * NC + lax.axis_index("c")


def _idx_body(xt_hbm, idx_hbm, x_v, idx_v):
    wid = _wid()

    pltpu.sync_copy(xt_hbm.at[:, pl.ds(wid * B_W, B_W)], x_v)

    lanes = lax.iota(jnp.int32, L)
    zeros = jnp.zeros((L,), jnp.int32)

    def off_f(f, _):
        f_vec = zeros + f
        for t in range(NW):
            vals = plsc.load_gather(x_v, [f_vec, t * L + lanes]) + f * F_DIM
            idx_v.at[f * 4 + (t // 8)][pl.ds((t % 8) * L, L)] = vals
        return 0

    lax.fori_loop(0, N_FIELDS, off_f, 0)

    pltpu.sync_copy(idx_v, idx_hbm.at[pl.ds(wid * N_DMA_ROWS, N_DMA_ROWS)])


def _gather_body(t0_hbm, t1_hbm, idx_hbm, out_hbm, idx0_v, idx1_v, mask_v,
                 rows0_v, rows1_v, out_v, sem):
    wid = _wid()

    pltpu.sync_copy(idx_hbm.at[pl.ds(wid * N_DMA_ROWS, N_DMA_ROWS)], idx0_v)

    ones_f = jnp.ones((L,), jnp.float32)
    zeros_f = jnp.zeros((L,), jnp.float32)

    def prep(j, _):
        for c in range(8):
            sl = pl.ds(c * L, L)
            iv = idx0_v.at[j][sl]
            m = iv < SPLIT
            idx0_v.at[j][sl] = jnp.where(
                m, iv, jnp.minimum(iv - SPLIT, P0_LEN - 1)
            )
            idx1_v.at[j][sl] = jnp.where(
                m, jnp.minimum(iv, P1_LEN - 1), iv - P1_START
            )
            mask_v.at[j][sl] = jnp.where(m, ones_f, zeros_f)
        return 0

    lax.fori_loop(0, N_DMA_ROWS, prep, 0)

    def fire(j, _):
        pltpu.make_async_copy(
            t0_hbm.at[idx0_v.at[j]], rows0_v.at[j], sem
        ).start()
        pltpu.make_async_copy(
            t1_hbm.at[idx1_v.at[j]], rows1_v.at[j], sem
        ).start()
        return 0

    lax.fori_loop(0, N_DMA_ROWS, fire, 0)

    def drain(j, _):
        pltpu.make_async_copy(
            t0_hbm.at[idx0_v.at[j]], rows0_v.at[j], sem
        ).wait()
        pltpu.make_async_copy(
            t1_hbm.at[idx1_v.at[j]], rows1_v.at[j], sem
        ).wait()
        return 0

    lax.fori_loop(0, N_DMA_ROWS, drain, 0)

    def reduce_t(t, _):
        g = t >> 3
        sl = pl.ds((t & 7) * L, L)
        acc = jnp.zeros((L,), jnp.float32)
        for f in range(N_FIELDS):
            r = f * 4 + g
            r0 = rows0_v.at[r][sl]
            r1 = rows1_v.at[r][sl]
            acc = acc + (r1 + (r0 - r1) * mask_v.at[r][sl])
        out_v[pl.ds(t * L, L)] = acc
        return 0

    lax.fori_loop(0, NW, reduce_t, 0)

    pltpu.sync_copy(out_v, out_hbm.at[pl.ds(wid * B_W, B_W)])


@jax.jit
def _features_linear(x, table):
    mesh = plsc.VectorSubcoreMesh(core_axis_name="c", subcore_axis_name="s")
    params = pltpu.CompilerParams(needs_layout_passes=False)

    idx_all = pl.kernel(
        _idx_body,
        mesh=mesh,
        compiler_params=params,
        out_type=jax.ShapeDtypeStruct((NW * N_DMA_ROWS, 128), jnp.int32),
        scratch_types=[
            pltpu.VMEM((N_FIELDS, B_W), jnp.int32),
            pltpu.VMEM((N_DMA_ROWS, 128), jnp.int32),
        ],
    )(jnp.transpose(x, (1, 0)))

    return pl.kernel(
        _gather_body,
        mesh=mesh,
        compiler_params=params,
        out_type=jax.ShapeDtypeStruct((B_TOTAL,), jnp.float32),
        scratch_types=[
            pltpu.VMEM((N_DMA_ROWS, 128), jnp.int32),
            pltpu.VMEM((N_DMA_ROWS, 128), jnp.int32),
            pltpu.VMEM((N_DMA_ROWS, 128), jnp.float32),
            pltpu.VMEM((N_DMA_ROWS, 128), jnp.float32),
            pltpu.VMEM((N_DMA_ROWS, 128), jnp.float32),
            pltpu.VMEM((B_W,), jnp.float32),
            pltpu.SemaphoreType.DMA,
        ],
    )(table[:P0_LEN, 0], table[P1_START:, 0], idx_all)


def kernel(x, table, bias):
    out = _features_linear(x, table)
    return out.reshape(B_TOTAL, 1) + bias


# speedup vs baseline: 29.5182x; 1.0210x over previous
import jax
import jax.numpy as jnp
from jax import lax
from jax.experimental import pallas as pl
from jax.experimental.pallas import tpu as pltpu
from jax.experimental.pallas import tpu_sc as plsc

N_FIELDS = 26
F_DIM = 100000
B_TOTAL = 16384
ROWS = N_FIELDS * F_DIM

P0_LEN = 1299456
P1_START = 1298496
P1_LEN = ROWS - P1_START
SPLIT = P0_LEN

_info = plsc.get_sparse_core_info()
NC, NS, L = _info.num_cores, _info.num_subcores, _info.num_lanes
NW = NC * NS
B_W = B_TOTAL // NW
IDX_W = B_W * N_FIELDS
N_DMA_ROWS = IDX_W // 128


def _wid():
    return lax.axis_index("s") * NC + lax.axis_index("c")


def _idx_body(xt_hbm, idx_hbm, x_v, idx_v):
    wid = _wid()

    pltpu.sync_copy(xt_hbm.at[:, pl.ds(wid * B_W, B_W)], x_v)

    lanes = lax.iota(jnp.int32, L)
    zeros = jnp.zeros((L,), jnp.int32)

    def off_f(f, _):
        f_vec = zeros + f
        for t in range(NW):
            vals = plsc.load_gather(x_v, [f_vec, t * L + lanes]) + f * F_DIM
            idx_v.at[f * 4 + (t // 8)][pl.ds((t % 8) * L, L)] = vals
        return 0

    lax.fori_loop(0, N_FIELDS, off_f, 0)

    pltpu.sync_copy(idx_v, idx_hbm.at[pl.ds(wid * N_DMA_ROWS, N_DMA_ROWS)])


def _gather_body(t0_hbm, t1_hbm, idx_hbm, out_hbm, idx0_v, idx1_v, mask_v,
                 rows0_v, rows1_v, out_v, sem):
    wid = _wid()

    pltpu.sync_copy(idx_hbm.at[pl.ds(wid * N_DMA_ROWS, N_DMA_ROWS)], idx0_v)

    ones_f = jnp.ones((L,), jnp.float32)
    zeros_f = jnp.zeros((L,), jnp.float32)

    def prep(j, _):
        for c in range(8):
            sl = pl.ds(c * L, L)
            iv = idx0_v.at[j][sl]
            m = iv < SPLIT
            idx0_v.at[j][sl] = jnp.where(
                m, iv, jnp.minimum(iv - SPLIT, P0_LEN - 1)
            )
            idx1_v.at[j][sl] = jnp.where(
                m, jnp.minimum(iv, P1_LEN - 1), iv - P1_START
            )
            mask_v.at[j][sl] = jnp.where(m, ones_f, zeros_f)
        pltpu.make_async_copy(
            t0_hbm.at[idx0_v.at[j]], rows0_v.at[j], sem
        ).start()
        pltpu.make_async_copy(
            t1_hbm.at[idx1_v.at[j]], rows1_v.at[j], sem
        ).start()
        return 0

    lax.fori_loop(0, N_DMA_ROWS, prep, 0)

    def drain(j, _):
        pltpu.make_async_copy(
            t0_hbm.at[idx0_v.at[j]], rows0_v.at[j], sem
        ).wait()
        pltpu.make_async_copy(
            t1_hbm.at[idx1_v.at[j]], rows1_v.at[j], sem
        ).wait()
        return 0

    lax.fori_loop(0, N_DMA_ROWS, drain, 0)

    def reduce_t(t, _):
        g = t >> 3
        sl = pl.ds((t & 7) * L, L)
        acc = jnp.zeros((L,), jnp.float32)
        for f in range(N_FIELDS):
            r = f * 4 + g
            r0 = rows0_v.at[r][sl]
            r1 = rows1_v.at[r][sl]
            acc = acc + (r1 + (r0 - r1) * mask_v.at[r][sl])
        out_v[pl.ds(t * L, L)] = acc
        return 0

    lax.fori_loop(0, NW, reduce_t, 0)

    pltpu.sync_copy(out_v, out_hbm.at[pl.ds(wid * B_W, B_W)])


@jax.jit
def _features_linear(x, table):
    mesh = plsc.VectorSubcoreMesh(core_axis_name="c", subcore_axis_name="s")
    params = pltpu.CompilerParams(needs_layout_passes=False)

    idx_all = pl.kernel(
        _idx_body,
        mesh=mesh,
        compiler_params=params,
        out_type=jax.ShapeDtypeStruct((NW * N_DMA_ROWS, 128), jnp.int32),
        scratch_types=[
            pltpu.VMEM((N_FIELDS, B_W), jnp.int32),
            pltpu.VMEM((N_DMA_ROWS, 128), jnp.int32),
        ],
    )(jnp.transpose(x, (1, 0)))

    return pl.kernel(
        _gather_body,
        mesh=mesh,
        compiler_params=params,
        out_type=jax.ShapeDtypeStruct((B_TOTAL,), jnp.float32),
        scratch_types=[
            pltpu.VMEM((N_DMA_ROWS, 128), jnp.int32),
            pltpu.VMEM((N_DMA_ROWS, 128), jnp.int32),
            pltpu.VMEM((N_DMA_ROWS, 128), jnp.float32),
            pltpu.VMEM((N_DMA_ROWS, 128), jnp.float32),
            pltpu.VMEM((N_DMA_ROWS, 128), jnp.float32),
            pltpu.VMEM((B_W,), jnp.float32),
            pltpu.SemaphoreType.DMA,
        ],
    )(table[:P0_LEN, 0], table[P1_START:, 0], idx_all)


def kernel(x, table, bias):
    out = _features_linear(x, table)
    return out.reshape(B_TOTAL, 1) + bias
